# baseline (speedup 1.0000x reference)
import functools
import jax
import jax.numpy as jnp
from jax import lax
from jax.experimental import pallas as pl
from jax.experimental.pallas import tpu as pltpu
from jax.experimental.pallas import tpu_sc as plsc

D = 64
SCALE = 8.0
NC = 2
NS = 16
NW = NC * NS
LANES = 16
C = 128
NB = 5
FB = D // 8


def _make_kernel(L: int, NBB: int):
  per_w = L
  assert NBB == NW and per_w % NB == 0

  mesh = plsc.VectorSubcoreMesh(core_axis_name="c", subcore_axis_name="s")

  @functools.partial(
      pl.kernel,
      mesh=mesh,
      compiler_params=pltpu.CompilerParams(
          use_tc_tiling_on_sc=False, needs_layout_passes=False),
      out_type=jax.ShapeDtypeStruct((L, FB, NBB, 8, C), jnp.float32),
      scratch_types=[
          pltpu.VMEM((per_w, C), jnp.int32),
          pltpu.VMEM((NB, C, D), jnp.float32),
          pltpu.VMEM((NB, D, C + 1), jnp.float32),
          [pltpu.SemaphoreType.DMA] * NB,
          [pltpu.SemaphoreType.DMA] * NB,
      ],
  )
  def body(x_hbm, table_hbm, out_hbm, idx_v, gbuf, tbuf, gsems, ssems):
    wid = lax.axis_index("s") * NC + lax.axis_index("c")
    pltpu.sync_copy(x_hbm.at[:, pl.ds(wid * C, C)], idx_v)

    def start_gather(g, b):
      pltpu.async_copy(table_hbm.at[idx_v.at[g]], gbuf.at[b], gsems[b])

    def wait_gather(b):
      pltpu.make_async_copy(table_hbm.at[idx_v.at[0]], gbuf.at[b],
                            gsems[b]).wait()

    def start_store(g, b):
      for fb in range(FB):
        pltpu.async_copy(tbuf.at[b, pl.ds(fb * 8, 8), pl.ds(0, C)],
                         out_hbm.at[g, fb, wid], ssems[b])

    def wait_store(b):
      for fb in range(FB):
        pltpu.make_async_copy(tbuf.at[b, pl.ds(fb * 8, 8), pl.ds(0, C)],
                              out_hbm.at[0, 0, 0], ssems[b]).wait()

    for b in range(NB):
      start_gather(b, b)

    fvecs = [lax.iota(jnp.int32, LANES) + (k * LANES) for k in range(D // LANES)]

    def unit(g, b):
      wait_gather(b)

      @pl.when(g >= NB)
      def _():
        wait_store(b)

      @plsc.parallel_loop(0, C, unroll=4)
      def _(i):
        col = jnp.full((LANES,), i, jnp.int32)
        for k in range(D // LANES):
          v = gbuf[b, i, pl.ds(k * LANES, LANES)]
          plsc.store_scatter(tbuf.at[b], [fvecs[k], col], v * SCALE)

      start_store(g, b)

      @pl.when(g + NB < per_w)
      def _():
        start_gather(g + NB, b)

    def outer(t, carry):
      for b in range(NB):
        unit(t + b, b)
      return carry

    lax.fori_loop(0, per_w // NB, lambda t, c: outer(t * NB, c), 0)

    for b in range(NB):
      wait_store(b)

  return body


def kernel(x, table):
  B, L = x.shape
  n = B * L
  nbb = B // C
  xu = x.astype(jnp.int32).T
  out5 = _make_kernel(L, nbb)(xu, table)
  return out5.transpose(2, 4, 0, 1, 3).reshape(B, L, D)


# speedup vs baseline: 1.0875x; 1.0875x over previous
import functools
import jax
import jax.numpy as jnp
from jax import lax
from jax.experimental import pallas as pl
from jax.experimental.pallas import tpu as pltpu
from jax.experimental.pallas import tpu_sc as plsc

D = 64
SCALE = 8.0
NC = 2
NS = 16
NW = NC * NS
LANES = 16
C = 128
NB = 4
FB = D // 8


def _make_kernel(L: int, NBB: int):
  per_w = L
  assert NBB == NW and per_w % NB == 0

  mesh = plsc.VectorSubcoreMesh(core_axis_name="c", subcore_axis_name="s")

  @functools.partial(
      pl.kernel,
      mesh=mesh,
      compiler_params=pltpu.CompilerParams(
          use_tc_tiling_on_sc=False, needs_layout_passes=False),
      out_type=jax.ShapeDtypeStruct((L, FB, NBB, 8, C), jnp.float32),
      scratch_types=[
          pltpu.VMEM((per_w, C), jnp.int32),
          pltpu.VMEM((NB, C, D), jnp.float32),
          pltpu.VMEM((NB, D, C + 1), jnp.float32),
          [pltpu.SemaphoreType.DMA] * NB,
          [pltpu.SemaphoreType.DMA] * NB,
      ],
  )
  def body(x_hbm, table_hbm, out_hbm, idx_v, gbuf, tbuf, gsems, ssems):
    wid = lax.axis_index("s") * NC + lax.axis_index("c")
    pltpu.sync_copy(x_hbm.at[:, pl.ds(wid * C, C)], idx_v)

    def start_gather(g, b):
      pltpu.async_copy(table_hbm.at[idx_v.at[g]], gbuf.at[b], gsems[b])

    def wait_gather(b):
      pltpu.make_async_copy(table_hbm.at[idx_v.at[0]], gbuf.at[b],
                            gsems[b]).wait()

    def start_store(g, b):
      for fb in range(FB):
        pltpu.async_copy(tbuf.at[b, pl.ds(fb * 8, 8), pl.ds(0, C)],
                         out_hbm.at[g, fb, wid], ssems[b])

    def wait_store(b):
      for fb in range(FB):
        pltpu.make_async_copy(tbuf.at[b, pl.ds(fb * 8, 8), pl.ds(0, C)],
                              out_hbm.at[0, 0, 0], ssems[b]).wait()

    for b in range(NB):
      start_gather(b, b)

    fvecs = [lax.iota(jnp.int32, LANES) + (k * LANES) for k in range(D // LANES)]

    def unit(g, b):
      wait_gather(b)

      @pl.when(g >= NB)
      def _():
        wait_store(b)

      @plsc.parallel_loop(0, C, unroll=4)
      def _(i):
        col = jnp.full((LANES,), i, jnp.int32)
        for k in range(D // LANES):
          v = gbuf[b, i, pl.ds(k * LANES, LANES)]
          plsc.store_scatter(tbuf.at[b], [fvecs[k], col], v * SCALE)

      start_store(g, b)

      @pl.when(g + NB < per_w)
      def _():
        start_gather(g + NB, b)

    def outer(t, carry):
      for b in range(NB):
        unit(t + b, b)
      return carry

    lax.fori_loop(0, per_w // NB, lambda t, c: outer(t * NB, c), 0)

    for b in range(NB):
      wait_store(b)

  return body


V_BLK = 2048


def _row_major_table(table):
  V, Dm = table.shape
  tt = table.T
  grid = (V + V_BLK - 1) // V_BLK

  def body(in_ref, out_ref):
    a = in_ref[...]
    at = a.T.reshape(V_BLK // 2, 2, Dm)
    out_ref[...] = jnp.concatenate([at[:, 0, :], at[:, 1, :]], axis=1)

  t2 = pl.pallas_call(
      body,
      grid=(grid,),
      in_specs=[pl.BlockSpec((Dm, V_BLK), lambda k: (0, k))],
      out_specs=pl.BlockSpec((V_BLK // 2, 2 * Dm), lambda k: (k, 0)),
      out_shape=jax.ShapeDtypeStruct((V // 2, 2 * Dm), jnp.float32),
  )(tt)
  return t2.reshape(V, Dm)


def kernel(x, table):
  B, L = x.shape
  n = B * L
  nbb = B // C
  xu = x.astype(jnp.int32).T
  out5 = _make_kernel(L, nbb)(xu, _row_major_table(table))
  return out5.transpose(2, 4, 0, 1, 3).reshape(B, L, D)
